# Initial kernel scaffold; baseline (speedup 1.0000x reference)
import jax
import jax.numpy as jnp
from jax.experimental import pallas as pl


def kernel(inputs, w, b):
    raise NotImplementedError("write your pallas kernel here")


# speedup vs baseline: 1.1891x; 1.1891x over previous
import functools

import jax
import jax.numpy as jnp
from jax import lax
from jax.experimental import pallas as pl
from jax.experimental.pallas import tpu as pltpu
from jax.experimental.pallas import tpu_sc as plsc

B = 16384
F = 26
NC, NS, L = 2, 16, 16
NW = NC * NS
RPW = B // NW
IPW = RPW * F
CH = IPW // 128

_mesh = plsc.VectorSubcoreMesh(core_axis_name="c", subcore_axis_name="s")


@functools.partial(
    pl.kernel,
    out_type=jax.ShapeDtypeStruct((B,), jnp.float32),
    mesh=_mesh,
    compiler_params=pltpu.CompilerParams(needs_layout_passes=False),
    scratch_types=[
        pltpu.VMEM((CH, 128), jnp.int32),
        pltpu.VMEM((IPW,), jnp.float32),
        pltpu.VMEM((RPW,), jnp.float32),
        pltpu.VMEM((L,), jnp.float32),
        pltpu.SemaphoreType.DMA,
    ],
)
def _lr_kernel(idx_hbm, w_hbm, b_hbm, out_hbm, idx_v, vals_v, out_v, b_v, sem):
    wid = lax.axis_index("s") * NC + lax.axis_index("c")
    pltpu.sync_copy(idx_hbm.at[wid], idx_v)
    pltpu.sync_copy(b_hbm, b_v)
    def fire_drain(s, carry):
        copies = []
        for c in range(8):
            j = s * 8 + c
            copies.append(
                pltpu.async_copy(
                    w_hbm.at[idx_v.at[j]], vals_v.at[pl.ds(j * 128, 128)], sem
                )
            )
        for cp in copies:
            cp.wait()
        return carry

    lax.fori_loop(0, CH // 8, fire_drain, 0)

    lanes = lax.broadcasted_iota(jnp.int32, (L,), 0)
    p0 = lanes * F
    bias = b_v[...]

    def group(g, carry):
        acc = bias
        pg = p0 + g * (L * F)
        for f in range(F):
            acc = acc + plsc.load_gather(vals_v, [pg + f])
        out_v[pl.ds(g * L, L)] = acc
        return carry

    lax.fori_loop(0, RPW // L, group, 0)
    pltpu.sync_copy(out_v, out_hbm.at[pl.ds(wid * RPW, RPW)])


def kernel(inputs, w, b):
    idx = inputs.reshape(NW, CH, 128)
    w_flat = w.reshape(-1)
    b_vec = jnp.broadcast_to(b, (L,)).astype(jnp.float32)
    out = _lr_kernel(idx, w_flat, b_vec)
    return out.reshape(B, 1)


# speedup vs baseline: 1.2925x; 1.0869x over previous
import functools

import jax
import jax.numpy as jnp
from jax import lax
from jax.experimental import pallas as pl
from jax.experimental.pallas import tpu as pltpu
from jax.experimental.pallas import tpu_sc as plsc

B = 16384
F = 26
NC, NS, L = 2, 16, 16
NW = NC * NS
RPW = B // NW
IPW = RPW * F
CH = IPW // 128

_mesh = plsc.VectorSubcoreMesh(core_axis_name="c", subcore_axis_name="s")


@functools.partial(
    pl.kernel,
    out_type=jax.ShapeDtypeStruct((B,), jnp.float32),
    mesh=_mesh,
    compiler_params=pltpu.CompilerParams(needs_layout_passes=False),
    scratch_types=[
        pltpu.VMEM((CH, 128), jnp.int32),
        pltpu.VMEM((IPW,), jnp.float32),
        pltpu.VMEM((RPW,), jnp.float32),
        pltpu.VMEM((L,), jnp.float32),
        pltpu.SemaphoreType.DMA,
    ],
)
def _lr_kernel(idx_hbm, w_hbm, b_hbm, out_hbm, idx_v, vals_v, out_v, b_v, sem):
    wid = lax.axis_index("s") * NC + lax.axis_index("c")
    pltpu.sync_copy(idx_hbm.at[wid], idx_v)
    pltpu.sync_copy(b_hbm, b_v)
    def fire(s, carry):
        for c in range(8):
            j = s * 8 + c
            pltpu.async_copy(
                w_hbm.at[idx_v.at[j]], vals_v.at[pl.ds(j * 128, 128)], sem
            )
        return carry

    lax.fori_loop(0, CH // 8, fire, 0)
    pltpu.make_async_copy(w_hbm.at[pl.ds(0, IPW)], vals_v, sem).wait()

    lanes = lax.broadcasted_iota(jnp.int32, (L,), 0)
    p0 = lanes * F
    bias = b_v[...]

    def group(g, carry):
        acc = bias
        pg = p0 + g * (L * F)
        for f in range(F):
            acc = acc + plsc.load_gather(vals_v, [pg + f])
        out_v[pl.ds(g * L, L)] = acc
        return carry

    lax.fori_loop(0, RPW // L, group, 0)
    pltpu.sync_copy(out_v, out_hbm.at[pl.ds(wid * RPW, RPW)])


def kernel(inputs, w, b):
    idx = inputs.reshape(NW, CH, 128)
    w_flat = w.reshape(-1)
    b_vec = jnp.broadcast_to(b, (L,)).astype(jnp.float32)
    out = _lr_kernel(idx, w_flat, b_vec)
    return out.reshape(B, 1)


# speedup vs baseline: 1.3165x; 1.0186x over previous
import functools

import jax
import jax.numpy as jnp
from jax import lax
from jax.experimental import pallas as pl
from jax.experimental.pallas import tpu as pltpu
from jax.experimental.pallas import tpu_sc as plsc

B = 16384
F = 26
NC, NS, L = 2, 16, 16
NW = NC * NS
RPW = B // NW
IPW = RPW * F
CH = IPW // 128

_mesh = plsc.VectorSubcoreMesh(core_axis_name="c", subcore_axis_name="s")


@functools.partial(
    pl.kernel,
    out_type=jax.ShapeDtypeStruct((B,), jnp.float32),
    mesh=_mesh,
    compiler_params=pltpu.CompilerParams(needs_layout_passes=False),
    scratch_types=[
        pltpu.VMEM((IPW,), jnp.int32),
        pltpu.VMEM((IPW,), jnp.float32),
        pltpu.VMEM((RPW,), jnp.float32),
        pltpu.VMEM((L,), jnp.float32),
        pltpu.SemaphoreType.DMA,
    ],
)
def _lr_kernel(idx_hbm, w_hbm, b_hbm, out_hbm, idx_v, vals_v, out_v, b_v, sem):
    wid = lax.axis_index("s") * NC + lax.axis_index("c")
    pltpu.sync_copy(idx_hbm.at[wid], idx_v)
    pltpu.sync_copy(b_hbm, b_v)
    pltpu.async_copy(w_hbm.at[idx_v], vals_v, sem).wait()

    lanes = lax.broadcasted_iota(jnp.int32, (L,), 0)
    p0 = lanes * F
    bias = b_v[...]

    def group(g, carry):
        acc = bias
        pg = p0 + g * (L * F)
        for f in range(F):
            acc = acc + plsc.load_gather(vals_v, [pg + f])
        out_v[pl.ds(g * L, L)] = acc
        return carry

    lax.fori_loop(0, RPW // L, group, 0)
    pltpu.sync_copy(out_v, out_hbm.at[pl.ds(wid * RPW, RPW)])


def kernel(inputs, w, b):
    idx = inputs.reshape(NW, IPW)
    w_flat = w.reshape(-1)
    b_vec = jnp.broadcast_to(b, (L,)).astype(jnp.float32)
    out = _lr_kernel(idx, w_flat, b_vec)
    return out.reshape(B, 1)


# speedup vs baseline: 1.3606x; 1.0335x over previous
import functools

import jax
import jax.numpy as jnp
from jax import lax
from jax.experimental import pallas as pl
from jax.experimental.pallas import tpu as pltpu
from jax.experimental.pallas import tpu_sc as plsc

B = 16384
F = 26
NC, NS, L = 2, 16, 16
NW = NC * NS
RPW = B // NW
IPW = RPW * F
CH = IPW // 128

_mesh = plsc.VectorSubcoreMesh(core_axis_name="c", subcore_axis_name="s")


@functools.partial(
    pl.kernel,
    out_type=jax.ShapeDtypeStruct((B,), jnp.float32),
    mesh=_mesh,
    compiler_params=pltpu.CompilerParams(
        needs_layout_passes=False, use_tc_tiling_on_sc=False
    ),
    scratch_types=[
        pltpu.VMEM((IPW,), jnp.int32),
        pltpu.VMEM((IPW,), jnp.float32),
        pltpu.VMEM((RPW,), jnp.float32),
        pltpu.VMEM((L,), jnp.float32),
        pltpu.SemaphoreType.DMA,
    ],
)
def _lr_kernel(idx_hbm, w_hbm, b_hbm, out_hbm, idx_v, vals_v, out_v, b_v, sem):
    wid = lax.axis_index("s") * NC + lax.axis_index("c")
    pltpu.sync_copy(idx_hbm.at[wid], idx_v)
    pltpu.sync_copy(b_hbm, b_v)
    pltpu.async_copy(w_hbm.at[0].at[idx_v], vals_v, sem).wait()

    lanes = lax.broadcasted_iota(jnp.int32, (L,), 0)
    p0 = lanes * F
    zeros = jnp.zeros((L,), jnp.int32)
    bias = b_v[...]

    def group(g, carry):
        acc = bias
        pg = p0 + g * (L * F)
        for f in range(F):
            acc = acc + plsc.load_gather(vals_v, [pg + f])
        out_v[pl.ds(g * L, L)] = acc
        return carry

    lax.fori_loop(0, RPW // L, group, 0)
    pltpu.sync_copy(out_v, out_hbm.at[pl.ds(wid * RPW, RPW)])


def kernel(inputs, w, b):
    idx = inputs.reshape(NW, IPW)
    b_vec = jnp.broadcast_to(b, (L,)).astype(jnp.float32)
    out = _lr_kernel(idx, w.reshape(1, -1), b_vec)
    return out.reshape(B, 1)


# speedup vs baseline: 2.7153x; 1.9957x over previous
import functools

import jax
import jax.numpy as jnp
from jax import lax
from jax.experimental import pallas as pl
from jax.experimental.pallas import tpu as pltpu
from jax.experimental.pallas import tpu_sc as plsc

B = 16384
F = 26
FPAD = 32
INPUT_ROWS = 1000000
WPAD = 1000448
NC, NS, L = 2, 16, 16
NW = NC * NS
RPW = B // NW
IPW = RPW * F
CB = B // 128
CBW = RPW // 128

_mesh = plsc.VectorSubcoreMesh(core_axis_name="c", subcore_axis_name="s")


@functools.partial(
    pl.kernel,
    out_type=jax.ShapeDtypeStruct((B,), jnp.float32),
    mesh=_mesh,
    compiler_params=pltpu.CompilerParams(
        needs_layout_passes=False, use_tc_tiling_on_sc=False
    ),
    scratch_types=[
        pltpu.VMEM((FPAD // 8, CBW, 8, 128), jnp.int32),
        pltpu.VMEM((IPW,), jnp.float32),
        pltpu.VMEM((RPW,), jnp.float32),
        pltpu.VMEM((L,), jnp.float32),
        pltpu.SemaphoreType.DMA,
    ],
)
def _lr_kernel(idx_hbm, w_hbm, b_hbm, out_hbm, idx_v, vals_v, out_v, b_v, sem):
    wid = lax.axis_index("s") * NC + lax.axis_index("c")
    cb0 = wid * CBW
    pltpu.sync_copy(idx_hbm.at[:, pl.ds(cb0, CBW), :, :], idx_v)
    pltpu.sync_copy(b_hbm, b_v)

    def fire(f, carry):
        rb = f // 8
        rr = lax.rem(f, 8)
        for j in range(CBW):
            iv = idx_v.at[rb, j, rr, :]
            pltpu.async_copy(
                w_hbm.at[iv], vals_v.at[pl.ds(f * RPW + j * 128, 128)], sem
            )
        return carry

    lax.fori_loop(0, F, fire, 0)
    pltpu.make_async_copy(w_hbm.at[pl.ds(0, IPW)], vals_v, sem).wait()

    bias = b_v[...]

    def group(g, carry):
        acc = bias
        for f in range(F):
            acc = acc + vals_v[pl.ds(f * RPW + g * L, L)]
        out_v[pl.ds(g * L, L)] = acc
        return carry

    lax.fori_loop(0, RPW // L, group, 0)
    pltpu.sync_copy(out_v, out_hbm.at[pl.ds(wid * RPW, RPW)])


def kernel(inputs, w, b):
    it = lax.pad(inputs.T, jnp.int32(0), ((0, FPAD - F, 0), (0, 0, 0)))
    idx = it.reshape(FPAD // 8, 8, CB, 128).transpose(0, 2, 1, 3)
    w_flat = lax.pad(w, jnp.float32(0), ((0, WPAD - INPUT_ROWS, 0), (0, 0, 0))).reshape(WPAD)
    b_vec = jnp.broadcast_to(b, (L,)).astype(jnp.float32)
    out = _lr_kernel(idx, w_flat, b_vec)
    return out.reshape(B, 1)
